# baseline (speedup 1.0000x reference)
import jax
import jax.numpy as jnp
from jax.experimental import pallas as pl
from jax.experimental.pallas import tpu as pltpu

_VOCAB = 1000000
_DIM = 64
_CTX = 200
_BN = 81920
_NB = -(-_VOCAB // _BN)


def _main_kernel(idx_ref, embt_hbm, mask_ref, wt_ref, b_ref, out_ref, lse_ref,
                 m_ref, cols_ref, stat_ref, sem):
    i = pl.program_id(0)

    @pl.when(i == 0)
    def _():
        def issue(j, c):
            base = pl.multiple_of(idx_ref[j] // 128 * 128, 128)
            pltpu.make_async_copy(
                embt_hbm.at[:, pl.ds(base, 128)],
                cols_ref.at[:, pl.ds(pl.multiple_of(128 * j, 128), 128)],
                sem).start()
            return c

        jax.lax.fori_loop(0, _CTX, issue, 0)

        def wait(j, c):
            base = pl.multiple_of(idx_ref[j] // 128 * 128, 128)
            pltpu.make_async_copy(
                embt_hbm.at[:, pl.ds(base, 128)],
                cols_ref.at[:, pl.ds(pl.multiple_of(128 * j, 128), 128)],
                sem).wait()
            return c

        jax.lax.fori_loop(0, _CTX, wait, 0)
        m_ref[...] = jnp.sum(cols_ref[...] * mask_ref[...], axis=1,
                             keepdims=True) * (1.0 / _CTX)
        stat_ref[0] = -jnp.inf
        stat_ref[1] = 0.0

    s = jax.lax.dot_general(
        m_ref[...], wt_ref[...], (((0,), (0,)), ((), ())),
        preferred_element_type=jnp.float32,
    ) + b_ref[...]
    out_ref[...] = s

    col = _BN * i + jax.lax.broadcasted_iota(jnp.int32, (1, _BN), 1)
    sm = jnp.where(col < _VOCAB, s, -jnp.inf)
    old_max = stat_ref[0]
    new_max = jnp.maximum(old_max, jnp.max(sm))
    stat_ref[1] = stat_ref[1] * jnp.exp(old_max - new_max) + jnp.sum(
        jnp.where(col < _VOCAB, jnp.exp(sm - new_max), 0.0))
    stat_ref[0] = new_max

    @pl.when(i == _NB - 1)
    def _():
        lse_ref[0, 0] = stat_ref[0] + jnp.log(stat_ref[1])


def _sub_kernel(x_ref, lse_ref, o_ref):
    o_ref[...] = x_ref[...] - lse_ref[0, 0]


@jax.jit
def kernel(inputs, emb_table, W, b):
    idx = inputs.astype(jnp.int32)
    onehot = (idx[:, None] % 128 ==
              jnp.arange(128, dtype=jnp.int32)[None, :]).astype(jnp.float32)
    mask = onehot.reshape(1, _CTX * 128)

    logits, lse = pl.pallas_call(
        _main_kernel,
        grid_spec=pltpu.PrefetchScalarGridSpec(
            num_scalar_prefetch=1,
            grid=(_NB,),
            in_specs=[
                pl.BlockSpec(memory_space=pl.ANY),
                pl.BlockSpec((1, _CTX * 128), lambda i, idx_ref: (0, 0)),
                pl.BlockSpec((_DIM, _BN), lambda i, idx_ref: (0, i)),
                pl.BlockSpec((1, _BN), lambda i, idx_ref: (0, i)),
            ],
            out_specs=[
                pl.BlockSpec((1, _BN), lambda i, idx_ref: (0, i)),
                pl.BlockSpec(memory_space=pltpu.SMEM),
            ],
            scratch_shapes=[
                pltpu.VMEM((_DIM, 1), jnp.float32),
                pltpu.VMEM((_DIM, _CTX * 128), jnp.float32),
                pltpu.SMEM((2,), jnp.float32),
                pltpu.SemaphoreType.DMA,
            ],
        ),
        out_shape=[
            jax.ShapeDtypeStruct((1, _VOCAB), jnp.float32),
            jax.ShapeDtypeStruct((1, 1), jnp.float32),
        ],
    )(idx, emb_table.T, mask, W.T, b.reshape(1, _VOCAB))

    out = pl.pallas_call(
        _sub_kernel,
        grid=(_NB,),
        in_specs=[
            pl.BlockSpec((1, _BN), lambda i: (0, i)),
            pl.BlockSpec(memory_space=pltpu.SMEM),
        ],
        out_specs=pl.BlockSpec((1, _BN), lambda i: (0, i)),
        out_shape=jax.ShapeDtypeStruct((1, _VOCAB), jnp.float32),
    )(logits, lse)

    return out
